# baseline (speedup 1.0000x reference)
import functools

import jax
import jax.numpy as jnp
from jax import lax
from jax.experimental import pallas as pl
from jax.experimental.pallas import tpu as pltpu
from jax.experimental.pallas import tpu_sc as plsc

NUM_GROUP = 100000
EMBED_DIM = 64
BATCH = 16384
HIST = 50

_INFO = plsc.get_sparse_core_info()
NC = _INFO.num_cores
NS = _INFO.num_subcores
NW = NC * NS

B = BATCH * HIST
CHUNK = 128
N_CHUNK_TOT = B // CHUNK
N_CHUNK = N_CHUNK_TOT // NW
BT_PER_H = BATCH // CHUNK
NBUF = 4
DT = EMBED_DIM // 8


def _body(idx_hbm, table_hbm, out_hbm, junk_hbm, idx_v, rows_v, tile_v, gsem, ssem):
    c = lax.axis_index("c")
    s = lax.axis_index("s")
    wid = s * NC + c
    cbase = wid * N_CHUNK

    pltpu.sync_copy(idx_hbm.at[wid], idx_v)

    lane = lax.iota(jnp.int32, 16)

    for b in range(NBUF):
        pltpu.async_copy(table_hbm.at[idx_v.at[b]], rows_v.at[b], gsem)
        for dt in range(DT):
            pltpu.async_copy(
                tile_v.at[b].at[pl.ds(dt * 1024, 1024)], junk_hbm, ssem
            )

    def step(i, carry):
        buf = lax.rem(i, NBUF)
        pltpu.make_async_copy(
            table_hbm.at[idx_v.at[i]], rows_v.at[buf], gsem
        ).wait()

        for dt in range(DT):
            pltpu.make_async_copy(
                tile_v.at[0].at[pl.ds(0, 1024)], junk_hbm, ssem
            ).wait()

        @plsc.parallel_loop(0, (CHUNK // 16) * EMBED_DIM, unroll=16)
        def _(j):
            bsg = j // EMBED_DIM
            d = j % EMBED_DIM
            rows16 = bsg * 16 + lane
            cols16 = jnp.full((16,), 1, jnp.int32) * d
            v = plsc.load_gather(rows_v.at[buf], [rows16, cols16])
            dt = d // 8
            ds = d % 8
            tile_v.at[buf][pl.ds(dt * 1024 + ds * 128 + bsg * 16, 16)] = v

        chunk = cbase + i
        h = chunk // BT_PER_H
        bt = chunk % BT_PER_H
        for dt in range(DT):
            pltpu.async_copy(
                tile_v.at[buf].at[pl.ds(dt * 1024, 1024)],
                out_hbm.at[h, dt, bt],
                ssem,
            )

        nxt = jnp.minimum(i + NBUF, N_CHUNK - 1)
        pltpu.async_copy(table_hbm.at[idx_v.at[nxt]], rows_v.at[buf], gsem)

        return carry

    lax.fori_loop(0, N_CHUNK, step, 0)

    for b in range(NBUF):
        pltpu.make_async_copy(
            table_hbm.at[idx_v.at[0]], rows_v.at[b], gsem
        ).wait()
        for dt in range(DT):
            pltpu.make_async_copy(
                tile_v.at[0].at[pl.ds(0, 1024)], junk_hbm, ssem
            ).wait()


@jax.jit
def _lookup(idx, table):
    kern = pl.kernel(
        _body,
        out_type=(
            jax.ShapeDtypeStruct((HIST, DT, BT_PER_H, 1024), jnp.float32),
            jax.ShapeDtypeStruct((1024,), jnp.float32),
        ),
        mesh=plsc.VectorSubcoreMesh(core_axis_name="c", subcore_axis_name="s"),
        scratch_types=[
            pltpu.VMEM((N_CHUNK, CHUNK), jnp.int32),
            pltpu.VMEM((NBUF, CHUNK, EMBED_DIM), jnp.float32),
            pltpu.VMEM((NBUF, DT * 1024), jnp.float32),
            pltpu.SemaphoreType.DMA,
            pltpu.SemaphoreType.DMA,
        ],
        compiler_params=pltpu.CompilerParams(
            use_tc_tiling_on_sc=False, needs_layout_passes=False
        ),
    )
    out4, _ = kern(idx, table)
    return out4


def kernel(num_group, table):
    idx = num_group.astype(jnp.int32).T.reshape(NW, N_CHUNK, CHUNK)
    out4 = _lookup(idx, table)
    out5 = out4.reshape(HIST, DT, BT_PER_H, 8, 128)
    return out5.transpose(2, 4, 0, 1, 3).reshape(BATCH, HIST, EMBED_DIM)


# speedup vs baseline: 1.1451x; 1.1451x over previous
import functools

import jax
import jax.numpy as jnp
from jax import lax
from jax.experimental import pallas as pl
from jax.experimental.pallas import tpu as pltpu
from jax.experimental.pallas import tpu_sc as plsc

NUM_GROUP = 100000
EMBED_DIM = 64
BATCH = 16384
HIST = 50

_INFO = plsc.get_sparse_core_info()
NC = _INFO.num_cores
NS = _INFO.num_subcores
NW = NC * NS

B = BATCH * HIST
CHUNK = 128
N_CHUNK_TOT = B // CHUNK
N_CHUNK = N_CHUNK_TOT // NW
BT_PER_H = BATCH // CHUNK
NBUF = 4
DT = EMBED_DIM // 8


def _body(idx_hbm, table_hbm, out_hbm, junk_hbm, idx_v, rows_v, tile_v, gsem, ssem):
    c = lax.axis_index("c")
    s = lax.axis_index("s")
    wid = s * NC + c
    cbase = wid * N_CHUNK

    pltpu.sync_copy(idx_hbm.at[wid], idx_v)

    lane = lax.iota(jnp.int32, 16)

    for b in range(NBUF):
        pltpu.async_copy(table_hbm.at[idx_v.at[b]], rows_v.at[b], gsem)
        pltpu.async_copy(tile_v.at[b], junk_hbm, ssem)

    def step(i, carry):
        buf = lax.rem(i, NBUF)
        pltpu.make_async_copy(
            table_hbm.at[idx_v.at[i]], rows_v.at[buf], gsem
        ).wait()

        pltpu.make_async_copy(tile_v.at[0], junk_hbm, ssem).wait()

        @plsc.parallel_loop(0, (CHUNK // 16) * EMBED_DIM, unroll=8)
        def _(j):
            bsg = j // EMBED_DIM
            d = j % EMBED_DIM
            rows16 = bsg * 16 + lane
            cols16 = jnp.full((16,), 1, jnp.int32) * d
            v = plsc.load_gather(rows_v.at[buf], [rows16, cols16])
            dt = d // 8
            ds = d % 8
            tile_v.at[buf][dt, pl.ds(ds * 128 + bsg * 16, 16)] = v

        chunk = cbase + i
        h = chunk // BT_PER_H
        bt = chunk % BT_PER_H
        pltpu.async_copy(tile_v.at[buf], out_hbm.at[h, :, bt], ssem)

        nxt = jnp.minimum(i + NBUF, N_CHUNK - 1)
        pltpu.async_copy(table_hbm.at[idx_v.at[nxt]], rows_v.at[buf], gsem)

        return carry

    lax.fori_loop(0, N_CHUNK, step, 0)

    for b in range(NBUF):
        pltpu.make_async_copy(
            table_hbm.at[idx_v.at[0]], rows_v.at[b], gsem
        ).wait()
        pltpu.make_async_copy(tile_v.at[0], junk_hbm, ssem).wait()


@jax.jit
def _lookup(idx, table):
    kern = pl.kernel(
        _body,
        out_type=(
            jax.ShapeDtypeStruct((HIST, DT, BT_PER_H, 1024), jnp.float32),
            jax.ShapeDtypeStruct((DT, 1024), jnp.float32),
        ),
        mesh=plsc.VectorSubcoreMesh(core_axis_name="c", subcore_axis_name="s"),
        scratch_types=[
            pltpu.VMEM((N_CHUNK, CHUNK), jnp.int32),
            pltpu.VMEM((NBUF, CHUNK, EMBED_DIM), jnp.float32),
            pltpu.VMEM((NBUF, DT, 1024), jnp.float32),
            pltpu.SemaphoreType.DMA,
            pltpu.SemaphoreType.DMA,
        ],
        compiler_params=pltpu.CompilerParams(
            use_tc_tiling_on_sc=False, needs_layout_passes=False
        ),
    )
    out4, _ = kern(idx, table)
    return out4


def kernel(num_group, table):
    idx = num_group.astype(jnp.int32).T.reshape(NW, N_CHUNK, CHUNK)
    out4 = _lookup(idx, table)
    out5 = out4.reshape(HIST, DT, BT_PER_H, 8, 128)
    return out5.transpose(2, 4, 0, 1, 3).reshape(BATCH, HIST, EMBED_DIM)
